# baseline (speedup 1.0000x reference)
import functools

import jax
import jax.numpy as jnp
from jax import lax
from jax.experimental import pallas as pl
from jax.experimental.pallas import tpu as pltpu
from jax.experimental.pallas import tpu_sc as plsc

N = 10000
E = 320000
F = 128
H = 128
C = 10
G = 64

NC = 2
NS = 16
NW = NC * NS

K = 128
CPW = 80
EP = NW * CPW * K
TOT = EP // K
SPLIT_C0 = TOT // 2
NP = 10240
RPT = NP // NS
BLK = 1024
NBLK = NP // BLK


def _mesh():
    return plsc.VectorSubcoreMesh(
        core_axis_name="c", subcore_axis_name="s",
        num_cores=NC, num_subcores=NS)


def _make_seg_kernel(gather: bool, W: int = 128):
    def body(*refs):
        if gather:
            (h_hbm, src_hbm, dst_hbm, out_hbm,
             sidx0, sidx1, didx0, didx1, rows0, rows1,
             accum, sem0, sem1, sems0, sems1) = refs
        else:
            (dst_hbm, out_hbm,
             sidx0, sidx1, didx0, didx1, rows0, rows1,
             accum, sem0, sem1, sems0, sems1) = refs
        sidx = (sidx0, sidx1)
        didx = (didx0, didx1)
        rows = (rows0, rows1)
        sem = (sem0, sem1)
        sems = (sems0, sems1)
        c = lax.axis_index("c")
        s = lax.axis_index("s")
        wid = c * NS + s
        tb = s * RPT

        z16 = jnp.zeros((16,), jnp.float32)

        @pl.loop(0, K)
        def _(i):
            for j in range(W // 16):
                rows0[i, pl.ds(j * 16, 16)] = z16

        for k in range(RPT // K):
            pltpu.sync_copy(rows0, accum.at[pl.ds(tb + k * K, K)])

        if not gather:
            o16 = jnp.ones((16,), jnp.float32)

            @pl.loop(0, K)
            def _(i):
                for j in range(W // 16):
                    rows0[i, pl.ds(j * 16, 16)] = o16
                    rows1[i, pl.ds(j * 16, 16)] = o16

        plsc.subcore_barrier()

        if gather:
            n0, n1 = SPLIT_C0 // NS, (TOT - SPLIT_C0) // NS
            n = jnp.where(c == 0, n0, n1)
            cbase = c * SPLIT_C0 + s * n
        else:
            n = CPW
            cbase = wid * CPW

        def fire_gather(chunk, b):
            base = (cbase + chunk) * K
            pltpu.sync_copy(dst_hbm.at[pl.ds(base, K)], didx[b])
            if gather:
                pltpu.sync_copy(src_hbm.at[pl.ds(base, K)], sidx[b])
                pltpu.async_copy(h_hbm.at[sidx[b]], rows[b], sem[b])

        def wait_gather(b):
            if gather:
                pltpu.make_async_copy(h_hbm.at[sidx[b]], rows[b], sem[b]).wait()

        def fire_scatter(b):
            pltpu.async_copy(rows[b], accum.at[didx[b]], sems[b], add=True)

        def wait_scatter(b):
            pltpu.make_async_copy(rows[b], accum.at[didx[b]], sems[b]).wait()

        @pl.when(n > 0)
        def _():
            fire_gather(0, 0)

        @pl.loop(0, n, step=2)
        def _(i):
            for b in range(2):
                cur = i + b
                nxt = cur + 1

                @pl.when((nxt < n) & (cur > 0))
                def _():
                    wait_scatter(1 - b)

                @pl.when(nxt < n)
                def _():
                    fire_gather(nxt, 1 - b)

                wait_gather(b)
                fire_scatter(b)

        @pl.when(n > 0)
        def _():
            wait_scatter(0)
            wait_scatter(1)

        plsc.subcore_barrier()

        pltpu.sync_copy(accum.at[pl.ds(tb, RPT)],
                        out_hbm.at[pl.ds(c * NP + tb, RPT)])

    return pl.kernel(
        body,
        out_type=jax.ShapeDtypeStruct((NC * NP, W), jnp.float32),
        mesh=_mesh(),
        scratch_types=[
            pltpu.VMEM((K,), jnp.int32),
            pltpu.VMEM((K,), jnp.int32),
            pltpu.VMEM((K,), jnp.int32),
            pltpu.VMEM((K,), jnp.int32),
            pltpu.VMEM((K, W), jnp.float32),
            pltpu.VMEM((K, W), jnp.float32),
            pltpu.VMEM_SHARED((NP, W), jnp.float32),
            pltpu.SemaphoreType.DMA,
            pltpu.SemaphoreType.DMA,
            pltpu.SemaphoreType.DMA,
            pltpu.SemaphoreType.DMA,
        ],
    )


def _make_cnt_kernel(w: int):
    def body(dst_hbm, cnt_hbm, didx, onesb, cntacc, sem):
        del sem
        c = lax.axis_index("c")
        s = lax.axis_index("s")
        wid = c * NS + s
        tb = s * RPT

        o16 = jnp.ones((16,), jnp.float32)

        @pl.loop(0, K)
        def _(i):
            for j in range(w // 16):
                onesb[i, pl.ds(j * 16, 16)] = o16

        @pl.loop(0, K)
        def _(i):
            for j in range(w // 16):
                onesb[i, pl.ds(j * 16, 16)] = jnp.zeros((16,), jnp.float32)

        for k in range(RPT // K):
            pltpu.sync_copy(onesb, cntacc.at[pl.ds(tb + k * K, K)])

        @pl.loop(0, K)
        def _(i):
            for j in range(w // 16):
                onesb[i, pl.ds(j * 16, 16)] = o16

        plsc.subcore_barrier()

        @pl.loop(0, CPW)
        def _(i):
            base = (wid * CPW + i) * K
            pltpu.sync_copy(dst_hbm.at[pl.ds(base, K)], didx)
            pltpu.sync_copy(onesb, cntacc.at[didx], add=True)

        plsc.subcore_barrier()
        pltpu.sync_copy(cntacc.at[pl.ds(tb, RPT)],
                        cnt_hbm.at[pl.ds(c * NP + tb, RPT)])

    return pl.kernel(
        body,
        out_type=jax.ShapeDtypeStruct((NC * NP, w), jnp.float32),
        mesh=_mesh(),
        scratch_types=[
            pltpu.VMEM((K,), jnp.int32),
            pltpu.VMEM((K, w), jnp.float32),
            pltpu.VMEM_SHARED((NP, w), jnp.float32),
            pltpu.SemaphoreType.DMA,
        ],
    )


@functools.lru_cache(maxsize=None)
def _seg_kernels():
    return _make_seg_kernel(True), _make_seg_kernel(False)


def _combine_body(p0, p1, c0, c1, h, wl, wr, b, out):
    cnt = c0[:, :1] + c1[:, :1]
    scale = 1.0 / jnp.maximum(cnt, 1.0)
    aggr = (p0[...] + p1[...]) * scale
    acc = jnp.dot(aggr, wl[...], preferred_element_type=jnp.float32)
    acc += jnp.dot(h[...], wr[...], preferred_element_type=jnp.float32)
    out[...] = jnp.maximum(acc + b[...], 0.0)


_combine = pl.pallas_call(
    _combine_body,
    grid=(NBLK,),
    in_specs=[
        pl.BlockSpec((BLK, 128), lambda i: (i, 0)),
        pl.BlockSpec((BLK, 128), lambda i: (i, 0)),
        pl.BlockSpec((BLK, 16), lambda i: (i, 0)),
        pl.BlockSpec((BLK, 16), lambda i: (i, 0)),
        pl.BlockSpec((BLK, 128), lambda i: (i, 0)),
        pl.BlockSpec((128, 128), lambda i: (0, 0)),
        pl.BlockSpec((128, 128), lambda i: (0, 0)),
        pl.BlockSpec((1, 128), lambda i: (0, 0)),
    ],
    out_specs=pl.BlockSpec((BLK, 128), lambda i: (i, 0)),
    out_shape=jax.ShapeDtypeStruct((NP, 128), jnp.float32),
)


def _final_body(h, batch, w4, b4, w5, b5, out, pooled, cnt):
    i = pl.program_id(0)

    @pl.when(i == 0)
    def _():
        pooled[...] = jnp.zeros_like(pooled)
        cnt[...] = jnp.zeros_like(cnt)

    bb = batch[...].reshape(1, BLK)
    gids = lax.broadcasted_iota(jnp.int32, (G, BLK), 0)
    mask = (bb == gids).astype(jnp.float32)
    pooled[...] += jnp.dot(mask, h[...], preferred_element_type=jnp.float32)
    cnt[...] += jnp.sum(mask, axis=1, keepdims=True)

    @pl.when(i == NBLK - 1)
    def _():
        mean = pooled[...] / jnp.maximum(cnt[:, :1], 1.0)
        t = jnp.maximum(
            jnp.dot(mean, w4[...], preferred_element_type=jnp.float32) + b4[...], 0.0)
        logits = jnp.dot(t, w5[...], preferred_element_type=jnp.float32) + b5[...]
        m = jnp.max(logits, axis=-1, keepdims=True)
        lse = jnp.log(jnp.sum(jnp.exp(logits - m), axis=-1, keepdims=True)) + m
        out[...] = logits - lse


_final = pl.pallas_call(
    _final_body,
    grid=(NBLK,),
    in_specs=[
        pl.BlockSpec((BLK, 128), lambda i: (i, 0)),
        pl.BlockSpec((1, 1, BLK), lambda i: (i, 0, 0)),
        pl.BlockSpec((128, 64), lambda i: (0, 0)),
        pl.BlockSpec((1, 64), lambda i: (0, 0)),
        pl.BlockSpec((64, C), lambda i: (0, 0)),
        pl.BlockSpec((1, C), lambda i: (0, 0)),
    ],
    out_specs=pl.BlockSpec((G, C), lambda i: (0, 0)),
    out_shape=jax.ShapeDtypeStruct((G, C), jnp.float32),
    scratch_shapes=[
        pltpu.VMEM((G, 128), jnp.float32),
        pltpu.VMEM((G, 1), jnp.float32),
    ],
)


def kernel(x, edge_index, batch, wl1, wr1, b1, wl2, wr2, b2, wl3, wr3, b3,
           w4, b4, w5, b5):
    src = edge_index[0]
    dst = edge_index[1]
    pad = jnp.arange(EP - E, dtype=jnp.int32)
    src_p = jnp.concatenate([src, pad % N])
    dst_p = jnp.concatenate([dst, N + pad % (NP - N)])
    h = jnp.concatenate([x, jnp.zeros((NP - N, F), jnp.float32)])
    batch_p = jnp.concatenate(
        [batch, jnp.full((NP - N,), G, jnp.int32)]).reshape(NBLK, 1, BLK)

    _seg, _cnt = _seg_kernels()
    cnts = _cnt(dst_p)
    c0, c1 = cnts[:NP, :16], cnts[NP:, :16]
    parts = _seg(h, src_p, dst_p)
    h = _combine(parts[:NP], parts[NP:], c0, c1, h, wl1, wr1, b1.reshape(1, H))

    parts = _seg(h, src_p, dst_p)
    h = _combine(parts[:NP], parts[NP:], c0, c1, h, wl2, wr2, b2.reshape(1, H))

    parts = _seg(h, src_p, dst_p)
    h = _combine(parts[:NP], parts[NP:], c0, c1, h, wl3, wr3, b3.reshape(1, H))

    return _final(h, batch_p, w4, b4.reshape(1, 64), w5, b5.reshape(1, C))


# speedup vs baseline: 1.0015x; 1.0015x over previous
import functools

import jax
import jax.numpy as jnp
from jax import lax
from jax.experimental import pallas as pl
from jax.experimental.pallas import tpu as pltpu
from jax.experimental.pallas import tpu_sc as plsc

N = 10000
E = 320000
F = 128
H = 128
C = 10
G = 64

NC = 2
NS = 16
NW = NC * NS

K = 128
CPW = 80
EP = NW * CPW * K
TOT = EP // K
SPLIT_C0 = TOT // 2
NP = 10240
RPT = NP // NS
BLK = 1024
NBLK = NP // BLK


def _mesh():
    return plsc.VectorSubcoreMesh(
        core_axis_name="c", subcore_axis_name="s",
        num_cores=NC, num_subcores=NS)


def _make_seg_kernel(gather: bool, W: int = 128):
    def body(*refs):
        if gather:
            (h_hbm, src_hbm, dst_hbm, out_hbm,
             sidx0, sidx1, didx0, didx1, rows0, rows1,
             accum, sem0, sem1, sems0, sems1) = refs
        else:
            (dst_hbm, out_hbm,
             sidx0, sidx1, didx0, didx1, rows0, rows1,
             accum, sem0, sem1, sems0, sems1) = refs
        sidx = (sidx0, sidx1)
        didx = (didx0, didx1)
        rows = (rows0, rows1)
        sem = (sem0, sem1)
        sems = (sems0, sems1)
        c = lax.axis_index("c")
        s = lax.axis_index("s")
        wid = c * NS + s
        tb = s * RPT

        z16 = jnp.zeros((16,), jnp.float32)

        @pl.loop(0, K)
        def _(i):
            for j in range(W // 16):
                rows0[i, pl.ds(j * 16, 16)] = z16

        for k in range(RPT // K):
            pltpu.sync_copy(rows0, accum.at[pl.ds(tb + k * K, K)])

        if not gather:
            o16 = jnp.ones((16,), jnp.float32)

            @pl.loop(0, K)
            def _(i):
                for j in range(W // 16):
                    rows0[i, pl.ds(j * 16, 16)] = o16
                    rows1[i, pl.ds(j * 16, 16)] = o16

        plsc.subcore_barrier()

        if gather:
            n0, n1 = SPLIT_C0 // NS, (TOT - SPLIT_C0) // NS
            n = jnp.where(c == 0, n0, n1)
            cbase = c * SPLIT_C0 + s * n
        else:
            n = CPW
            cbase = wid * CPW

        def fire_gather(chunk, b):
            base = (cbase + chunk) * K
            pltpu.sync_copy(dst_hbm.at[pl.ds(base, K)], didx[b])
            if gather:
                pltpu.sync_copy(src_hbm.at[pl.ds(base, K)], sidx[b])
                pltpu.async_copy(h_hbm.at[sidx[b]], rows[b], sem[b])

        def wait_gather(b):
            if gather:
                pltpu.make_async_copy(h_hbm.at[sidx[b]], rows[b], sem[b]).wait()

        def fire_scatter(b):
            pltpu.async_copy(rows[b], accum.at[didx[b]], sems[b], add=True)

        def wait_scatter(b):
            pltpu.make_async_copy(rows[b], accum.at[didx[b]], sems[b]).wait()

        @pl.when(n > 0)
        def _():
            fire_gather(0, 0)

        @pl.loop(0, n, step=2)
        def _(i):
            for b in range(2):
                cur = i + b
                nxt = cur + 1

                @pl.when((nxt < n) & (cur > 0))
                def _():
                    wait_scatter(1 - b)

                @pl.when(nxt < n)
                def _():
                    fire_gather(nxt, 1 - b)

                wait_gather(b)
                fire_scatter(b)

        @pl.when(n > 0)
        def _():
            wait_scatter(0)
            wait_scatter(1)

        plsc.subcore_barrier()

        pltpu.sync_copy(accum.at[pl.ds(tb, RPT)],
                        out_hbm.at[pl.ds(c * NP + tb, RPT)])

    return pl.kernel(
        body,
        out_type=jax.ShapeDtypeStruct((NC * NP, W), jnp.float32),
        mesh=_mesh(),
        scratch_types=[
            pltpu.VMEM((K,), jnp.int32),
            pltpu.VMEM((K,), jnp.int32),
            pltpu.VMEM((K,), jnp.int32),
            pltpu.VMEM((K,), jnp.int32),
            pltpu.VMEM((K, W), jnp.float32),
            pltpu.VMEM((K, W), jnp.float32),
            pltpu.VMEM_SHARED((NP, W), jnp.float32),
            pltpu.SemaphoreType.DMA,
            pltpu.SemaphoreType.DMA,
            pltpu.SemaphoreType.DMA,
            pltpu.SemaphoreType.DMA,
        ],
    )


@functools.lru_cache(maxsize=None)
def _seg_kernels():
    return _make_seg_kernel(True), _make_seg_kernel(False)


def _combine_body(p0, p1, c0, c1, h, wl, wr, b, out):
    cnt = c0[:, :1] + c1[:, :1]
    scale = 1.0 / jnp.maximum(cnt, 1.0)
    aggr = (p0[...] + p1[...]) * scale
    acc = jnp.dot(aggr, wl[...], preferred_element_type=jnp.float32)
    acc += jnp.dot(h[...], wr[...], preferred_element_type=jnp.float32)
    out[...] = jnp.maximum(acc + b[...], 0.0)


_combine = pl.pallas_call(
    _combine_body,
    grid=(NBLK,),
    in_specs=[
        pl.BlockSpec((BLK, 128), lambda i: (i, 0)),
        pl.BlockSpec((BLK, 128), lambda i: (i, 0)),
        pl.BlockSpec((BLK, 16), lambda i: (i, 0)),
        pl.BlockSpec((BLK, 16), lambda i: (i, 0)),
        pl.BlockSpec((BLK, 128), lambda i: (i, 0)),
        pl.BlockSpec((128, 128), lambda i: (0, 0)),
        pl.BlockSpec((128, 128), lambda i: (0, 0)),
        pl.BlockSpec((1, 128), lambda i: (0, 0)),
    ],
    out_specs=pl.BlockSpec((BLK, 128), lambda i: (i, 0)),
    out_shape=jax.ShapeDtypeStruct((NP, 128), jnp.float32),
)


def _final_body(h, batch, w4, b4, w5, b5, out, pooled, cnt):
    i = pl.program_id(0)

    @pl.when(i == 0)
    def _():
        pooled[...] = jnp.zeros_like(pooled)
        cnt[...] = jnp.zeros_like(cnt)

    bb = batch[...].reshape(1, BLK)
    gids = lax.broadcasted_iota(jnp.int32, (G, BLK), 0)
    mask = (bb == gids).astype(jnp.float32)
    pooled[...] += jnp.dot(mask, h[...], preferred_element_type=jnp.float32)
    cnt[...] += jnp.sum(mask, axis=1, keepdims=True)

    @pl.when(i == NBLK - 1)
    def _():
        mean = pooled[...] / jnp.maximum(cnt[:, :1], 1.0)
        t = jnp.maximum(
            jnp.dot(mean, w4[...], preferred_element_type=jnp.float32) + b4[...], 0.0)
        logits = jnp.dot(t, w5[...], preferred_element_type=jnp.float32) + b5[...]
        m = jnp.max(logits, axis=-1, keepdims=True)
        lse = jnp.log(jnp.sum(jnp.exp(logits - m), axis=-1, keepdims=True)) + m
        out[...] = logits - lse


_final = pl.pallas_call(
    _final_body,
    grid=(NBLK,),
    in_specs=[
        pl.BlockSpec((BLK, 128), lambda i: (i, 0)),
        pl.BlockSpec((1, 1, BLK), lambda i: (i, 0, 0)),
        pl.BlockSpec((128, 64), lambda i: (0, 0)),
        pl.BlockSpec((1, 64), lambda i: (0, 0)),
        pl.BlockSpec((64, C), lambda i: (0, 0)),
        pl.BlockSpec((1, C), lambda i: (0, 0)),
    ],
    out_specs=pl.BlockSpec((G, C), lambda i: (0, 0)),
    out_shape=jax.ShapeDtypeStruct((G, C), jnp.float32),
    scratch_shapes=[
        pltpu.VMEM((G, 128), jnp.float32),
        pltpu.VMEM((G, 1), jnp.float32),
    ],
)


def kernel(x, edge_index, batch, wl1, wr1, b1, wl2, wr2, b2, wl3, wr3, b3,
           w4, b4, w5, b5):
    src = edge_index[0]
    dst = edge_index[1]
    pad = jnp.arange(EP - E, dtype=jnp.int32)
    src_p = jnp.concatenate([src, pad % N])
    dst_p = jnp.concatenate([dst, N + pad % (NP - N)])
    h = jnp.concatenate([x, jnp.zeros((NP - N, F), jnp.float32)])
    batch_p = jnp.concatenate(
        [batch, jnp.full((NP - N,), G, jnp.int32)]).reshape(NBLK, 1, BLK)

    _seg, _cnt = _seg_kernels()
    cnts = _cnt(dst_p)
    c0, c1 = cnts[:NP, :16], cnts[NP:, :16]
    parts = _seg(h, src_p, dst_p)
    h = _combine(parts[:NP], parts[NP:], c0, c1, h, wl1, wr1, b1.reshape(1, H))

    parts = _seg(h, src_p, dst_p)
    h = _combine(parts[:NP], parts[NP:], c0, c1, h, wl2, wr2, b2.reshape(1, H))

    parts = _seg(h, src_p, dst_p)
    h = _combine(parts[:NP], parts[NP:], c0, c1, h, wl3, wr3, b3.reshape(1, H))

    return _final(h, batch_p, w4, b4.reshape(1, 64), w5, b5.reshape(1, C))
